# Initial kernel scaffold; baseline (speedup 1.0000x reference)
import jax
import jax.numpy as jnp
from jax.experimental import pallas as pl


def kernel(a, v, l, dia_len, topicLabel, fc_a_w, fc_a_b, fc_v_w, fc_v_b, fc_l_w, fc_l_b, conv_w):
    raise NotImplementedError("write your pallas kernel here")


# speedup vs baseline: 44.4969x; 44.4969x over previous
import math

import jax
import jax.numpy as jnp
from jax.experimental import pallas as pl

_NLAYERS = 4
_LAMDA = 0.5
_ALPHA = 0.1
_EDGE = 0.99999


def _gcn_dialogue_kernel(a_ref, v_ref, l_ref, wa_ref, ba_ref, wv_ref,
                         bv_ref, wl_ref, bl_ref, conv_ref, out_ref):
    a = a_ref[0]
    v = v_ref[0]
    l = l_ref[0]

    def sim_block(x):
        vec_len = jnp.sqrt(jnp.sum(x * x, axis=1, keepdims=True))
        nt = x / vec_len
        cos = jax.lax.dot_general(nt, nt, (((1,), (1,)), ((), ())),
                                  preferred_element_type=jnp.float32) * _EDGE
        return jax.lax.atan2(jnp.sqrt(jnp.maximum(1.0 - cos * cos, 0.0)), cos)

    sims = [sim_block(a), sim_block(v), sim_block(l)]

    dis_col = []
    dis_row = []
    for s in sims:
        rs_c = jnp.sum(s, axis=1, keepdims=True) + 2.0 * _EDGE
        rs_r = jnp.sum(s, axis=0, keepdims=True) + 2.0 * _EDGE
        dis_col.append(rs_c ** -0.5)
        dis_row.append(rs_r ** -0.5)

    blocks = [dis_col[m] * sims[m] * dis_row[m] for m in range(3)]
    cross = [[dis_col[m] * _EDGE * dis_col[q] for q in range(3)]
             for m in range(3)]

    def fc(x, w_ref, b_ref):
        y = jax.lax.dot_general(x, w_ref[...], (((1,), (1,)), ((), ())),
                                preferred_element_type=jnp.float32)
        return jax.nn.relu(y + b_ref[...])

    h0 = [fc(a, wa_ref, ba_ref), fc(v, wv_ref, bv_ref), fc(l, wl_ref, bl_ref)]
    layer = list(h0)

    nhid = h0[0].shape[1]
    for i in range(_NLAYERS):
        theta = math.log(_LAMDA / (i + 1) + 1.0)
        w1 = conv_ref[i, :nhid, :]
        w2 = conv_ref[i, nhid:, :]
        new_layer = []
        for m in range(3):
            hi = jnp.dot(blocks[m], layer[m],
                         preferred_element_type=jnp.float32)
            for q in range(3):
                if q != m:
                    hi = hi + cross[m][q] * layer[q]
            mix = jnp.dot(hi, w1, preferred_element_type=jnp.float32) + \
                  jnp.dot(h0[m], w2, preferred_element_type=jnp.float32)
            r = (1.0 - _ALPHA) * hi + _ALPHA * h0[m]
            new_layer.append(jax.nn.relu(theta * mix + (1.0 - theta) * r))
        layer = new_layer

    out_ref[0] = jnp.concatenate([l, layer[0], layer[1], layer[2]], axis=1)


def kernel(a, v, l, dia_len, topicLabel, fc_a_w, fc_a_b, fc_v_w, fc_v_b,
           fc_l_w, fc_l_b, conv_w):
    n, nfeat = l.shape
    ndia = dia_len.shape[0]
    dlen = n // ndia
    nhid = fc_a_w.shape[0]

    a3 = a.reshape(ndia, dlen, nfeat)
    v3 = v.reshape(ndia, dlen, nfeat)
    l3 = l.reshape(ndia, dlen, nfeat)
    ba = fc_a_b.reshape(1, nhid)
    bv = fc_v_b.reshape(1, nhid)
    bl = fc_l_b.reshape(1, nhid)

    feat_spec = pl.BlockSpec((1, dlen, nfeat), lambda d: (d, 0, 0))
    w_spec = pl.BlockSpec((nhid, nfeat), lambda d: (0, 0))
    b_spec = pl.BlockSpec((1, nhid), lambda d: (0, 0))
    conv_spec = pl.BlockSpec(conv_w.shape, lambda d: (0, 0, 0))

    out = pl.pallas_call(
        _gcn_dialogue_kernel,
        grid=(ndia,),
        in_specs=[feat_spec, feat_spec, feat_spec, w_spec, b_spec, w_spec,
                  b_spec, w_spec, b_spec, conv_spec],
        out_specs=pl.BlockSpec((1, dlen, nfeat + 3 * nhid),
                               lambda d: (d, 0, 0)),
        out_shape=jax.ShapeDtypeStruct((ndia, dlen, nfeat + 3 * nhid),
                                       jnp.float32),
    )(a3, v3, l3, fc_a_w, ba, fc_v_w, bv, fc_l_w, bl, conv_w)

    return out.reshape(n, nfeat + 3 * nhid)


# speedup vs baseline: 50.6214x; 1.1376x over previous
import math

import jax
import jax.numpy as jnp
from jax.experimental import pallas as pl

_NLAYERS = 4
_LAMDA = 0.5
_ALPHA = 0.1
_EDGE = 0.99999
_DIAS_PER_STEP = 4


def _dialogue_out(a, v, l, wa_ref, ba_ref, wv_ref, bv_ref, wl_ref, bl_ref,
                  conv_ref):
    def sim_block(x):
        vec_len = jnp.sqrt(jnp.sum(x * x, axis=1, keepdims=True))
        nt = x / vec_len
        cos = jax.lax.dot_general(nt, nt, (((1,), (1,)), ((), ())),
                                  preferred_element_type=jnp.float32) * _EDGE
        return jax.lax.atan2(jnp.sqrt(jnp.maximum(1.0 - cos * cos, 0.0)), cos)

    sims = [sim_block(a), sim_block(v), sim_block(l)]

    dis_col = []
    dis_row = []
    for s in sims:
        rs_c = jnp.sum(s, axis=1, keepdims=True) + 2.0 * _EDGE
        rs_r = jnp.sum(s, axis=0, keepdims=True) + 2.0 * _EDGE
        dis_col.append(rs_c ** -0.5)
        dis_row.append(rs_r ** -0.5)

    blocks = [dis_col[m] * sims[m] * dis_row[m] for m in range(3)]
    cross = [[dis_col[m] * _EDGE * dis_col[q] for q in range(3)]
             for m in range(3)]

    def fc(x, w_ref, b_ref):
        y = jax.lax.dot_general(x, w_ref[...], (((1,), (1,)), ((), ())),
                                preferred_element_type=jnp.float32)
        return jax.nn.relu(y + b_ref[...])

    h0 = [fc(a, wa_ref, ba_ref), fc(v, wv_ref, bv_ref), fc(l, wl_ref, bl_ref)]
    layer = list(h0)

    for i in range(_NLAYERS):
        theta = math.log(_LAMDA / (i + 1) + 1.0)
        w = conv_ref[i]
        new_layer = []
        for m in range(3):
            hi = jnp.dot(blocks[m], layer[m],
                         preferred_element_type=jnp.float32)
            for q in range(3):
                if q != m:
                    hi = hi + cross[m][q] * layer[q]
            support = jnp.concatenate([hi, h0[m]], axis=1)
            mix = jnp.dot(support, w, preferred_element_type=jnp.float32)
            r = (1.0 - _ALPHA) * hi + _ALPHA * h0[m]
            new_layer.append(jax.nn.relu(theta * mix + (1.0 - theta) * r))
        layer = new_layer

    return jnp.concatenate([l, layer[0], layer[1], layer[2]], axis=1)


def _gcn_kernel(a_ref, v_ref, l_ref, wa_ref, ba_ref, wv_ref, bv_ref, wl_ref,
                bl_ref, conv_ref, out_ref):
    for j in range(_DIAS_PER_STEP):
        out_ref[j] = _dialogue_out(a_ref[j], v_ref[j], l_ref[j], wa_ref,
                                   ba_ref, wv_ref, bv_ref, wl_ref, bl_ref,
                                   conv_ref)


def kernel(a, v, l, dia_len, topicLabel, fc_a_w, fc_a_b, fc_v_w, fc_v_b,
           fc_l_w, fc_l_b, conv_w):
    n, nfeat = l.shape
    ndia = dia_len.shape[0]
    dlen = n // ndia
    nhid = fc_a_w.shape[0]
    grid = ndia // _DIAS_PER_STEP

    a3 = a.reshape(ndia, dlen, nfeat)
    v3 = v.reshape(ndia, dlen, nfeat)
    l3 = l.reshape(ndia, dlen, nfeat)
    ba = fc_a_b.reshape(1, nhid)
    bv = fc_v_b.reshape(1, nhid)
    bl = fc_l_b.reshape(1, nhid)

    feat_spec = pl.BlockSpec((_DIAS_PER_STEP, dlen, nfeat),
                             lambda d: (d, 0, 0))
    w_spec = pl.BlockSpec((nhid, nfeat), lambda d: (0, 0))
    b_spec = pl.BlockSpec((1, nhid), lambda d: (0, 0))
    conv_spec = pl.BlockSpec(conv_w.shape, lambda d: (0, 0, 0))

    out = pl.pallas_call(
        _gcn_kernel,
        grid=(grid,),
        in_specs=[feat_spec, feat_spec, feat_spec, w_spec, b_spec, w_spec,
                  b_spec, w_spec, b_spec, conv_spec],
        out_specs=pl.BlockSpec((_DIAS_PER_STEP, dlen, nfeat + 3 * nhid),
                               lambda d: (d, 0, 0)),
        out_shape=jax.ShapeDtypeStruct((ndia, dlen, nfeat + 3 * nhid),
                                       jnp.float32),
    )(a3, v3, l3, fc_a_w, ba, fc_v_w, bv, fc_l_w, bl, conv_w)

    return out.reshape(n, nfeat + 3 * nhid)


# speedup vs baseline: 50.7792x; 1.0031x over previous
import math

import jax
import jax.numpy as jnp
from jax.experimental import pallas as pl

_NLAYERS = 4
_LAMDA = 0.5
_ALPHA = 0.1
_EDGE = 0.99999
_DIAS_PER_STEP = 10


def _dialogue_out(a, v, l, wa_ref, ba_ref, wv_ref, bv_ref, wl_ref, bl_ref,
                  conv_ref):
    def sim_block(x):
        vec_len = jnp.sqrt(jnp.sum(x * x, axis=1, keepdims=True))
        nt = x / vec_len
        cos = jax.lax.dot_general(nt, nt, (((1,), (1,)), ((), ())),
                                  preferred_element_type=jnp.float32) * _EDGE
        return jax.lax.atan2(jnp.sqrt(jnp.maximum(1.0 - cos * cos, 0.0)), cos)

    sims = [sim_block(a), sim_block(v), sim_block(l)]

    dis_col = []
    dis_row = []
    for s in sims:
        rs_c = jnp.sum(s, axis=1, keepdims=True) + 2.0 * _EDGE
        rs_r = jnp.sum(s, axis=0, keepdims=True) + 2.0 * _EDGE
        dis_col.append(rs_c ** -0.5)
        dis_row.append(rs_r ** -0.5)

    blocks = [dis_col[m] * sims[m] * dis_row[m] for m in range(3)]
    cross = [[dis_col[m] * _EDGE * dis_col[q] for q in range(3)]
             for m in range(3)]

    def fc(x, w_ref, b_ref):
        y = jax.lax.dot_general(x, w_ref[...], (((1,), (1,)), ((), ())),
                                preferred_element_type=jnp.float32)
        return jax.nn.relu(y + b_ref[...])

    h0 = [fc(a, wa_ref, ba_ref), fc(v, wv_ref, bv_ref), fc(l, wl_ref, bl_ref)]
    layer = list(h0)

    for i in range(_NLAYERS):
        theta = math.log(_LAMDA / (i + 1) + 1.0)
        w = conv_ref[i]
        new_layer = []
        for m in range(3):
            hi = jnp.dot(blocks[m], layer[m],
                         preferred_element_type=jnp.float32)
            for q in range(3):
                if q != m:
                    hi = hi + cross[m][q] * layer[q]
            support = jnp.concatenate([hi, h0[m]], axis=1)
            mix = jnp.dot(support, w, preferred_element_type=jnp.float32)
            r = (1.0 - _ALPHA) * hi + _ALPHA * h0[m]
            new_layer.append(jax.nn.relu(theta * mix + (1.0 - theta) * r))
        layer = new_layer

    return jnp.concatenate([l, layer[0], layer[1], layer[2]], axis=1)


def _gcn_kernel(a_ref, v_ref, l_ref, wa_ref, ba_ref, wv_ref, bv_ref, wl_ref,
                bl_ref, conv_ref, out_ref):
    for j in range(_DIAS_PER_STEP):
        out_ref[j] = _dialogue_out(a_ref[j], v_ref[j], l_ref[j], wa_ref,
                                   ba_ref, wv_ref, bv_ref, wl_ref, bl_ref,
                                   conv_ref)


def kernel(a, v, l, dia_len, topicLabel, fc_a_w, fc_a_b, fc_v_w, fc_v_b,
           fc_l_w, fc_l_b, conv_w):
    n, nfeat = l.shape
    ndia = dia_len.shape[0]
    dlen = n // ndia
    nhid = fc_a_w.shape[0]
    grid = ndia // _DIAS_PER_STEP

    a3 = a.reshape(ndia, dlen, nfeat)
    v3 = v.reshape(ndia, dlen, nfeat)
    l3 = l.reshape(ndia, dlen, nfeat)
    ba = fc_a_b.reshape(1, nhid)
    bv = fc_v_b.reshape(1, nhid)
    bl = fc_l_b.reshape(1, nhid)

    feat_spec = pl.BlockSpec((_DIAS_PER_STEP, dlen, nfeat),
                             lambda d: (d, 0, 0))
    w_spec = pl.BlockSpec((nhid, nfeat), lambda d: (0, 0))
    b_spec = pl.BlockSpec((1, nhid), lambda d: (0, 0))
    conv_spec = pl.BlockSpec(conv_w.shape, lambda d: (0, 0, 0))

    out = pl.pallas_call(
        _gcn_kernel,
        grid=(grid,),
        in_specs=[feat_spec, feat_spec, feat_spec, w_spec, b_spec, w_spec,
                  b_spec, w_spec, b_spec, conv_spec],
        out_specs=pl.BlockSpec((_DIAS_PER_STEP, dlen, nfeat + 3 * nhid),
                               lambda d: (d, 0, 0)),
        out_shape=jax.ShapeDtypeStruct((ndia, dlen, nfeat + 3 * nhid),
                                       jnp.float32),
    )(a3, v3, l3, fc_a_w, ba, fc_v_w, bv, fc_l_w, bl, conv_w)

    return out.reshape(n, nfeat + 3 * nhid)


# speedup vs baseline: 71.0393x; 1.3990x over previous
import math

import jax
import jax.numpy as jnp
from jax.experimental import pallas as pl

_NLAYERS = 4
_LAMDA = 0.5
_ALPHA = 0.1
_EDGE = 0.99999
_DIAS_PER_STEP = 10

_ACOS_C = (1.5707963050, -0.2145988016, 0.0889789874, -0.0501743046,
           0.0308918810, -0.0170881256, 0.0066700901, -0.0012624911)


def _acos(x):
    ax = jnp.abs(x)
    p = _ACOS_C[7]
    for c in (_ACOS_C[6], _ACOS_C[5], _ACOS_C[4], _ACOS_C[3], _ACOS_C[2],
              _ACOS_C[1], _ACOS_C[0]):
        p = p * ax + c
    r = jnp.sqrt(1.0 - ax) * p
    return jnp.where(x < 0, math.pi - r, r)


def _gcn_kernel(a_ref, v_ref, l_ref, wa_ref, ba_ref, wv_ref, bv_ref, wl_ref,
                bl_ref, w12_ref, out_ref):
    a = a_ref[...]
    v = v_ref[...]
    l = l_ref[...]

    def sim_block(x):
        vec_len = jnp.sqrt(jnp.sum(x * x, axis=2, keepdims=True))
        nt = x / vec_len
        cos = jax.lax.dot_general(nt, nt, (((2,), (2,)), ((0,), (0,))),
                                  preferred_element_type=jnp.float32) * _EDGE
        return _acos(cos)

    sims = [sim_block(a), sim_block(v), sim_block(l)]

    dis_col = []
    dis_row = []
    for s in sims:
        rs_c = jnp.sum(s, axis=2, keepdims=True) + 2.0 * _EDGE
        rs_r = jnp.sum(s, axis=1, keepdims=True) + 2.0 * _EDGE
        dis_col.append(rs_c ** -0.5)
        dis_row.append(rs_r ** -0.5)

    blocks = [dis_col[m] * sims[m] * dis_row[m] for m in range(3)]
    cross = [[dis_col[m] * _EDGE * dis_col[q] for q in range(3)]
             for m in range(3)]

    def fc(x, w_ref, b_ref):
        y = jax.lax.dot_general(x, w_ref[...], (((2,), (1,)), ((), ())),
                                preferred_element_type=jnp.float32)
        return jax.nn.relu(y + b_ref[...])

    h0 = [fc(a, wa_ref, ba_ref), fc(v, wv_ref, bv_ref), fc(l, wl_ref, bl_ref)]
    layer = list(h0)

    for i in range(_NLAYERS):
        w12 = w12_ref[i]
        new_layer = []
        for m in range(3):
            hi = jax.lax.dot_general(blocks[m], layer[m],
                                     (((2,), (1,)), ((0,), (0,))),
                                     preferred_element_type=jnp.float32)
            for q in range(3):
                if q != m:
                    hi = hi + cross[m][q] * layer[q]
            support = jnp.concatenate([hi, h0[m]], axis=2)
            new_layer.append(jax.nn.relu(
                jax.lax.dot_general(support, w12, (((2,), (0,)), ((), ())),
                                    preferred_element_type=jnp.float32)))
        layer = new_layer

    out_ref[...] = jnp.concatenate([l, layer[0], layer[1], layer[2]], axis=2)


def kernel(a, v, l, dia_len, topicLabel, fc_a_w, fc_a_b, fc_v_w, fc_v_b,
           fc_l_w, fc_l_b, conv_w):
    n, nfeat = l.shape
    ndia = dia_len.shape[0]
    dlen = n // ndia
    nhid = fc_a_w.shape[0]
    grid = ndia // _DIAS_PER_STEP

    a3 = a.reshape(ndia, dlen, nfeat)
    v3 = v.reshape(ndia, dlen, nfeat)
    l3 = l.reshape(ndia, dlen, nfeat)
    ba = fc_a_b.reshape(1, nhid)
    bv = fc_v_b.reshape(1, nhid)
    bl = fc_l_b.reshape(1, nhid)

    eye = jnp.eye(nhid, dtype=jnp.float32)
    thetas = jnp.asarray(
        [math.log(_LAMDA / (i + 1) + 1.0) for i in range(_NLAYERS)],
        dtype=jnp.float32)[:, None, None]
    w1_eff = thetas * conv_w[:, :nhid, :] + \
        (1.0 - thetas) * (1.0 - _ALPHA) * eye[None]
    w2_eff = thetas * conv_w[:, nhid:, :] + \
        (1.0 - thetas) * _ALPHA * eye[None]
    w12_eff = jnp.concatenate([w1_eff, w2_eff], axis=1)

    feat_spec = pl.BlockSpec((_DIAS_PER_STEP, dlen, nfeat),
                             lambda d: (d, 0, 0))
    w_spec = pl.BlockSpec((nhid, nfeat), lambda d: (0, 0))
    b_spec = pl.BlockSpec((1, nhid), lambda d: (0, 0))
    conv_spec = pl.BlockSpec((_NLAYERS, 2 * nhid, nhid), lambda d: (0, 0, 0))

    out = pl.pallas_call(
        _gcn_kernel,
        grid=(grid,),
        in_specs=[feat_spec, feat_spec, feat_spec, w_spec, b_spec, w_spec,
                  b_spec, w_spec, b_spec, conv_spec],
        out_specs=pl.BlockSpec((_DIAS_PER_STEP, dlen, nfeat + 3 * nhid),
                               lambda d: (d, 0, 0)),
        out_shape=jax.ShapeDtypeStruct((ndia, dlen, nfeat + 3 * nhid),
                                       jnp.float32),
    )(a3, v3, l3, fc_a_w, ba, fc_v_w, bv, fc_l_w, bl, w12_eff)

    return out.reshape(n, nfeat + 3 * nhid)


# speedup vs baseline: 72.0066x; 1.0136x over previous
import math

import jax
import jax.numpy as jnp
from jax.experimental import pallas as pl

_NLAYERS = 4
_LAMDA = 0.5
_ALPHA = 0.1
_EDGE = 0.99999
_DIAS_PER_STEP = 5

_ACOS_C = (1.5707288, -0.2121144, 0.0742610, -0.0187293)


def _acos(x):
    ax = jnp.abs(x)
    p = _ACOS_C[3]
    for c in (_ACOS_C[2], _ACOS_C[1], _ACOS_C[0]):
        p = p * ax + c
    r = jnp.sqrt(1.0 - ax) * p
    return jnp.where(x < 0, math.pi - r, r)


def _bdot(x, y):
    return jax.lax.dot_general(x, y, (((2,), (1,)), ((0,), (0,))),
                               preferred_element_type=jnp.float32)


def _wdot(x, w):
    return jax.lax.dot_general(x, w, (((2,), (0,)), ((), ())),
                               preferred_element_type=jnp.float32)


def _gcn_kernel(a_ref, v_ref, l_ref, wa_ref, ba_ref, wv_ref, bv_ref, wl_ref,
                bl_ref, w12_ref, out_ref):
    a = a_ref[...]
    v = v_ref[...]
    l = l_ref[...]

    def sim_block(x):
        vec_len = jnp.sqrt(jnp.sum(x * x, axis=2, keepdims=True))
        nt = x / vec_len
        cos = jax.lax.dot_general(nt, nt, (((2,), (2,)), ((0,), (0,))),
                                  preferred_element_type=jnp.float32) * _EDGE
        return _acos(cos)

    sims = [sim_block(a), sim_block(v), sim_block(l)]

    dis = []
    dis2 = []
    for s in sims:
        rs = jnp.sum(s, axis=2, keepdims=True) + 2.0 * _EDGE
        dis.append(rs ** -0.5)
        dis2.append(1.0 / rs)

    def fc(x, w_ref, b_ref):
        y = jax.lax.dot_general(x, w_ref[...], (((2,), (1,)), ((), ())),
                                preferred_element_type=jnp.float32)
        return jax.nn.relu(y + b_ref[...])

    h0 = [fc(a, wa_ref, ba_ref), fc(v, wv_ref, bv_ref), fc(l, wl_ref, bl_ref)]
    z0 = [dis[m] * h0[m] for m in range(3)]
    z = list(z0)

    for i in range(_NLAYERS):
        last = i == _NLAYERS - 1
        w12 = w12_ref[i]
        zsum = z[0] + z[1] + z[2]
        new = []
        for m in range(3):
            u = _bdot(sims[m], z[m]) + _EDGE * (zsum - z[m])
            support = jnp.concatenate(
                [(dis[m] if last else dis2[m]) * u,
                 h0[m] if last else z0[m]], axis=2)
            new.append(jax.nn.relu(_wdot(support, w12)))
        z = new

    out_ref[...] = jnp.concatenate([l, z[0], z[1], z[2]], axis=2)


def kernel(a, v, l, dia_len, topicLabel, fc_a_w, fc_a_b, fc_v_w, fc_v_b,
           fc_l_w, fc_l_b, conv_w):
    n, nfeat = l.shape
    ndia = dia_len.shape[0]
    dlen = n // ndia
    nhid = fc_a_w.shape[0]
    grid = ndia // _DIAS_PER_STEP

    a3 = a.reshape(ndia, dlen, nfeat)
    v3 = v.reshape(ndia, dlen, nfeat)
    l3 = l.reshape(ndia, dlen, nfeat)
    ba = fc_a_b.reshape(1, nhid)
    bv = fc_v_b.reshape(1, nhid)
    bl = fc_l_b.reshape(1, nhid)

    eye = jnp.eye(nhid, dtype=jnp.float32)
    thetas = jnp.asarray(
        [math.log(_LAMDA / (i + 1) + 1.0) for i in range(_NLAYERS)],
        dtype=jnp.float32)[:, None, None]
    w1_eff = thetas * conv_w[:, :nhid, :] + \
        (1.0 - thetas) * (1.0 - _ALPHA) * eye[None]
    w2_eff = thetas * conv_w[:, nhid:, :] + \
        (1.0 - thetas) * _ALPHA * eye[None]
    w12_eff = jnp.concatenate([w1_eff, w2_eff], axis=1)

    feat_spec = pl.BlockSpec((_DIAS_PER_STEP, dlen, nfeat),
                             lambda d: (d, 0, 0))
    w_spec = pl.BlockSpec((nhid, nfeat), lambda d: (0, 0))
    b_spec = pl.BlockSpec((1, nhid), lambda d: (0, 0))
    conv_spec = pl.BlockSpec((_NLAYERS, 2 * nhid, nhid), lambda d: (0, 0, 0))

    out = pl.pallas_call(
        _gcn_kernel,
        grid=(grid,),
        in_specs=[feat_spec, feat_spec, feat_spec, w_spec, b_spec, w_spec,
                  b_spec, w_spec, b_spec, conv_spec],
        out_specs=pl.BlockSpec((_DIAS_PER_STEP, dlen, nfeat + 3 * nhid),
                               lambda d: (d, 0, 0)),
        out_shape=jax.ShapeDtypeStruct((ndia, dlen, nfeat + 3 * nhid),
                                       jnp.float32),
    )(a3, v3, l3, fc_a_w, ba, fc_v_w, bv, fc_l_w, bl, w12_eff)

    return out.reshape(n, nfeat + 3 * nhid)
